# baseline (speedup 1.0000x reference)
import functools

import jax
import jax.numpy as jnp
from jax import lax
from jax.experimental import pallas as pl
from jax.experimental.pallas import tpu as pltpu
from jax.experimental.pallas import tpu_sc as plsc

_N = 4096
_H = 128
_EXP = 8
_K = 2
_NE = 65536
_TAU = 1.0
_BN = 512
_SUB = 64
_EPS_LN = 1e-5

_NSC = 2
_NSUB = 16
_NTILE = _NSC * _NSUB
_EPT = _NE // _NTILE
_CHUNK = 128
_NCH = _EPT // _CHUNK
_RPT = _N // _NSUB
_W = 128

_INTERPRET = False


def _sc_deg_body(dst3, zeros_hbm, ones_hbm, out_hbm,
                 didx, onesv, stage, shared, ssem):
    c = lax.axis_index("c")
    s = lax.axis_index("s")
    gw = c * _NSUB + s
    pltpu.sync_copy(zeros_hbm.at[pl.ds(s * _RPT, _RPT)], stage)
    pltpu.sync_copy(stage, shared.at[pl.ds(s * _RPT, _RPT)])
    pltpu.sync_copy(ones_hbm, onesv)
    pltpu.sync_copy(dst3.at[gw], didx)
    plsc.subcore_barrier()
    cps = [pltpu.async_copy(onesv, shared.at[didx.at[j]], ssem, add=True)
           for j in range(_NCH)]
    for cp in cps:
        cp.wait()
    plsc.subcore_barrier()
    pltpu.sync_copy(shared.at[pl.ds(s * _RPT, _RPT)], stage)
    pltpu.sync_copy(stage, out_hbm.at[c, pl.ds(s * _RPT, _RPT)])


_NBUF = 4


def _sc_y_body(src3, dst3, xws_hbm, zeros_hbm, out_hbm,
               sidx, didx, gbuf, stage, shared, gsem, ssem):
    c = lax.axis_index("c")
    s = lax.axis_index("s")
    gw = c * _NSUB + s
    for r in range(_RPT // _CHUNK):
        base = s * _RPT + r * _CHUNK
        pltpu.sync_copy(zeros_hbm.at[pl.ds(base, _CHUNK)], stage)
        pltpu.sync_copy(stage, shared.at[pl.ds(base, _CHUNK)])
    pltpu.sync_copy(src3.at[gw], sidx)
    pltpu.sync_copy(dst3.at[gw], didx)
    plsc.subcore_barrier()
    gcps = [None] * _NCH
    scps = [None] * _NCH
    for j in range(min(_NBUF, _NCH)):
        gcps[j] = pltpu.async_copy(xws_hbm.at[sidx.at[j]], gbuf.at[j], gsem)
    for j in range(_NCH):
        if j >= _NBUF:
            scps[j - _NBUF].wait()
            gcps[j] = pltpu.async_copy(xws_hbm.at[sidx.at[j]],
                                       gbuf.at[j % _NBUF], gsem)
        gcps[j].wait()
        scps[j] = pltpu.async_copy(gbuf.at[j % _NBUF], shared.at[didx.at[j]],
                                   ssem, add=True)
    for j in range(max(_NCH - _NBUF, 0), _NCH):
        scps[j].wait()
    plsc.subcore_barrier()
    for r in range(_RPT // _CHUNK):
        base = s * _RPT + r * _CHUNK
        pltpu.sync_copy(shared.at[pl.ds(base, _CHUNK)], stage)
        pltpu.sync_copy(stage, out_hbm.at[c, pl.ds(base, _CHUNK)])


def _softmax(x):
    m = jnp.max(x, axis=-1, keepdims=True)
    e = jnp.exp(x - m)
    return e / jnp.sum(e, axis=-1, keepdims=True)


def _sigmoid(x):
    return 1.0 / (1.0 + jnp.exp(-x))


def _silu(x):
    return x * _sigmoid(x)


def _gate(score, g):
    return _sigmoid(score + _softmax((score + g[0]) / _TAU)
                    - _softmax((score + g[1]) / _TAU))


def _top2_weights(logits):
    cols = jax.lax.broadcasted_iota(jnp.int32, logits.shape, 1)
    v1 = jnp.max(logits, axis=1, keepdims=True)
    i1 = jnp.min(jnp.where(logits == v1, cols, _EXP), axis=1, keepdims=True)
    m1 = cols == i1
    l2 = jnp.where(m1, -jnp.inf, logits)
    v2 = jnp.max(l2, axis=1, keepdims=True)
    i2 = jnp.min(jnp.where(l2 == v2, cols, _EXP), axis=1, keepdims=True)
    m2 = cols == i2
    w1 = _sigmoid(v1 - v2)
    return jnp.where(m1, w1, 0.0) + jnp.where(m2, 1.0 - w1, 0.0)


def _scale_body(emb_x_ref, gcn_w_ref, deg_ref, xws_ref):
    xw = jnp.dot(emb_x_ref[...].astype(jnp.bfloat16),
                 gcn_w_ref[...].astype(jnp.bfloat16),
                 preferred_element_type=jnp.float32)
    deg = jnp.sum(deg_ref[...], axis=0)[:, 0:1] + 1.0
    xws_ref[...] = xw * jax.lax.rsqrt(deg)


def _main_body(emb_x_ref, emb_v_ref, y_ref, deg_ref, xws_ref, gcn_b_ref,
               gn_ref, lw_ref, lb_ref, w1cat_ref, b1cat_ref,
               exw2_ref, exb2_ref, exg_ref, exbeta_ref,
               evxw2_ref, evxb2_ref, evw_ref, evg_ref,
               ln2g_ref, ln2b_ref, lnv2g_ref,
               lx_ref, lv_ref, score_ref):
    x = emb_x_ref[...]
    v = emb_v_ref[...]
    deg = jnp.sum(deg_ref[...], axis=0)[:, 0:1] + 1.0
    dinv = jax.lax.rsqrt(deg)
    ysum = jnp.sum(y_ref[...], axis=0) + xws_ref[...]
    gcn = dinv * ysum[:, :_EXP] + gcn_b_ref[...][0:1, :_EXP]
    score_x = _softmax(gcn)
    gn = gn_ref[...]
    logits_x = _gate(score_x, (gn[:, 0:8], gn[:, 8:16]))

    svs = []
    for s in range(_BN // _SUB):
        bv = v[s * _SUB:(s + 1) * _SUB].astype(jnp.bfloat16) \
            .astype(jnp.float32)
        vs = (bv[:, 0, :, None] * bv[:, 0, None, :]
              + bv[:, 1, :, None] * bv[:, 1, None, :]
              + bv[:, 2, :, None] * bv[:, 2, None, :]).astype(jnp.bfloat16)
        svs.append(jax.lax.dot_general(
            vs.reshape(_SUB, _H * _H), lw_ref[...],
            (((1,), (0,)), ((), ())),
            preferred_element_type=jnp.float32))
    sv = jnp.concatenate(svs, axis=0) + lb_ref[...][0:1, :]
    score_v = _softmax(sv[:, :_EXP])
    logits_v = _gate(score_v, (gn[:, 16:24], gn[:, 24:32]))

    wx = _top2_weights(logits_x)
    wv = _top2_weights(logits_v)

    h0 = _silu(jnp.dot(x, w1cat_ref[...],
                       preferred_element_type=jnp.float32)
               + b1cat_ref[...][0:1, :])

    lx_acc = jnp.zeros((_BN, _H), jnp.float32)
    lv_acc = [jnp.zeros((_BN, _H), jnp.float32) for _ in range(3)]
    for e in range(_EXP):
        h1x = h0[:, e * _H:(e + 1) * _H]
        h2 = _silu(jnp.dot(h1x, exw2_ref[e],
                           preferred_element_type=jnp.float32)
                   + exb2_ref[e][None, :])
        mu = jnp.mean(h2, axis=1, keepdims=True)
        var = jnp.mean((h2 - mu) ** 2, axis=1, keepdims=True)
        aex = (h2 - mu) * jax.lax.rsqrt(var + _EPS_LN) \
            * exg_ref[e][None, :] + exbeta_ref[e][None, :]
        lx_acc = lx_acc + wx[:, e:e + 1] * aex

        h1v = h0[:, (_EXP + e) * _H:(_EXP + e + 1) * _H]
        hv2 = _silu(jnp.dot(h1v, evxw2_ref[e],
                            preferred_element_type=jnp.float32)
                    + evxb2_ref[e][None, :])
        ev = jnp.dot(hv2, evw_ref[e], preferred_element_type=jnp.float32)
        aev = ev[:, None, :] * v
        rms = jax.lax.rsqrt(
            jnp.mean(aev * aev, axis=(1, 2), keepdims=True) + _EPS_LN)
        aev = aev * rms * evg_ref[e][None, None, :]
        for k in range(3):
            c = e + _EXP * k
            lv_acc[c % 3] = lv_acc[c % 3] \
                + wv[:, c // 3:c // 3 + 1] * aev[:, k, :]

    lxs = lx_acc + x
    mu = jnp.mean(lxs, axis=1, keepdims=True)
    var = jnp.mean((lxs - mu) ** 2, axis=1, keepdims=True)
    lx_ref[...] = (lxs - mu) * jax.lax.rsqrt(var + _EPS_LN) \
        * ln2g_ref[...][0:1, :] + ln2b_ref[...][0:1, :]

    lvs = jnp.stack(lv_acc, axis=1) + v
    rms2 = jax.lax.rsqrt(
        jnp.mean(lvs * lvs, axis=(1, 2), keepdims=True) + _EPS_LN)
    lv_ref[...] = lvs * rms2 * lnv2g_ref[...][0:1, None, :]

    score_ref[...] = _softmax(
        jnp.concatenate([logits_x, logits_v], axis=1))


def _gumbel_noise():
    base = jax.random.key(42)
    gs = []
    for i in range(4):
        u = jax.random.uniform(jax.random.fold_in(base, i), (_N, _EXP),
                               minval=1e-6, maxval=1.0 - 1e-6)
        gs.append(-jnp.log(-jnp.log(u)))
    return jnp.concatenate(gs, axis=1)


def kernel(emb_x, emb_v, edge_index, gcn_W, gcn_b, lin_W, lin_b,
           exW1, exb1, exW2, exb2, exg, exbeta,
           evxW1, evxb1, evxW2, evxb2, evW, evg, ln2g, ln2b, lnv2g):
    src3 = edge_index[0].reshape(_NTILE, _NCH, _CHUNK)
    dst3 = edge_index[1].reshape(_NTILE, _NCH, _CHUNK)
    zeros_n = jnp.zeros((_N, _W), jnp.float32)
    ones_c = jnp.ones((_CHUNK, _W), jnp.float32)

    mesh = plsc.VectorSubcoreMesh(core_axis_name="c", subcore_axis_name="s")
    deg_p = pl.kernel(
        _sc_deg_body,
        out_type=jax.ShapeDtypeStruct((_NSC, _N, _W), jnp.float32),
        mesh=mesh,
        scratch_types=[
            pltpu.VMEM((_NCH, _CHUNK), jnp.int32),
            pltpu.VMEM((_CHUNK, _W), jnp.float32),
            pltpu.VMEM((_RPT, _W), jnp.float32),
            pltpu.VMEM_SHARED((_N, _W), jnp.float32),
            pltpu.SemaphoreType.DMA,
        ],
    )(dst3, zeros_n, ones_c)

    gcn_wp = jnp.pad(gcn_W, ((0, 0), (0, _W - _EXP)))
    xws = pl.pallas_call(
        _scale_body,
        grid=(1,),
        in_specs=[
            pl.BlockSpec((_N, _H), lambda i: (0, 0)),
            pl.BlockSpec((_H, _W), lambda i: (0, 0)),
            pl.BlockSpec((_NSC, _N, _W), lambda i: (0, 0, 0)),
        ],
        out_specs=pl.BlockSpec((_N, _W), lambda i: (0, 0)),
        out_shape=jax.ShapeDtypeStruct((_N, _W), jnp.float32),
        interpret=_INTERPRET,
    )(emb_x, gcn_wp, deg_p)

    y_p = pl.kernel(
        _sc_y_body,
        out_type=jax.ShapeDtypeStruct((_NSC, _N, _W), jnp.float32),
        mesh=mesh,
        scratch_types=[
            pltpu.VMEM((_NCH, _CHUNK), jnp.int32),
            pltpu.VMEM((_NCH, _CHUNK), jnp.int32),
            pltpu.VMEM((_NBUF, _CHUNK, _W), jnp.float32),
            pltpu.VMEM((_CHUNK, _W), jnp.float32),
            pltpu.VMEM_SHARED((_N, _W), jnp.float32),
            pltpu.SemaphoreType.DMA,
            pltpu.SemaphoreType.DMA,
        ],
    )(src3, dst3, xws, zeros_n)

    lin_wp = jnp.pad(lin_W, ((0, 0), (0, 16 - _EXP)))
    lin_bp = jnp.pad(lin_b, (0, 16 - _EXP))[None, :]
    gnoise = _gumbel_noise()
    w1cat = jnp.concatenate(
        [jnp.transpose(exW1, (1, 0, 2)).reshape(_H, _EXP * _H),
         jnp.transpose(evxW1, (1, 0, 2)).reshape(_H, _EXP * _H)], axis=1)
    b1cat = jnp.concatenate([exb1.reshape(-1), evxb1.reshape(-1)])[None, :]
    gcn_bp = jnp.pad(gcn_b, (0, 16 - _EXP))[None, :]

    grid = (_N // _BN,)
    full = lambda *s: pl.BlockSpec(s, lambda i: (0,) * len(s))
    lx, lv, score = pl.pallas_call(
        _main_body,
        grid=grid,
        in_specs=[
            pl.BlockSpec((_BN, _H), lambda i: (i, 0)),
            pl.BlockSpec((_BN, 3, _H), lambda i: (i, 0, 0)),
            pl.BlockSpec((_NSC, _BN, _W), lambda i: (0, i, 0)),
            pl.BlockSpec((_NSC, _BN, _W), lambda i: (0, i, 0)),
            pl.BlockSpec((_BN, _W), lambda i: (i, 0)),
            full(1, 16),
            pl.BlockSpec((_BN, 32), lambda i: (i, 0)),
            full(_H * _H, 16),
            full(1, 16),
            full(_H, 2 * _EXP * _H),
            full(1, 2 * _EXP * _H),
            full(_EXP, _H, _H),
            full(_EXP, _H),
            full(_EXP, _H),
            full(_EXP, _H),
            full(_EXP, _H, _H),
            full(_EXP, _H),
            full(_EXP, _H, _H),
            full(_EXP, _H),
            full(1, _H),
            full(1, _H),
            full(1, _H),
        ],
        out_specs=[
            pl.BlockSpec((_BN, _H), lambda i: (i, 0)),
            pl.BlockSpec((_BN, 3, _H), lambda i: (i, 0, 0)),
            pl.BlockSpec((_BN, 16), lambda i: (i, 0)),
        ],
        out_shape=[
            jax.ShapeDtypeStruct((_N, _H), jnp.float32),
            jax.ShapeDtypeStruct((_N, 3, _H), jnp.float32),
            jax.ShapeDtypeStruct((_N, 16), jnp.float32),
        ],
        interpret=_INTERPRET,
    )(emb_x, emb_v, y_p, deg_p, xws, gcn_bp, gnoise, lin_wp, lin_bp,
      w1cat, b1cat, exW2, exb2, exg, exbeta, evxW2, evxb2, evW, evg,
      ln2g[None, :], ln2b[None, :], lnv2g[None, :])

    return lx, lv, score[:, :2 * _EXP]


# speedup vs baseline: 1.4075x; 1.4075x over previous
import functools

import jax
import jax.numpy as jnp
from jax import lax
from jax.experimental import pallas as pl
from jax.experimental.pallas import tpu as pltpu
from jax.experimental.pallas import tpu_sc as plsc

_N = 4096
_H = 128
_EXP = 8
_K = 2
_NE = 65536
_TAU = 1.0
_BN = 256
_SUB = 64
_EPS_LN = 1e-5

_NSC = 2
_NSUB = 16
_NTILE = _NSC * _NSUB
_EPT = _NE // _NTILE
_CHUNK = 128
_NCH = _EPT // _CHUNK
_RPT = _N // _NSUB
_W = 128

_INTERPRET = False


def _sc_deg_body(dst3, zeros_hbm, ones_hbm, out_hbm,
                 didx, onesv, stage, shared, ssem):
    c = lax.axis_index("c")
    s = lax.axis_index("s")
    gw = c * _NSUB + s
    pltpu.sync_copy(zeros_hbm.at[pl.ds(s * _RPT, _RPT)], stage)
    pltpu.sync_copy(stage, shared.at[pl.ds(s * _RPT, _RPT)])
    pltpu.sync_copy(ones_hbm, onesv)
    pltpu.sync_copy(dst3.at[gw], didx)
    plsc.subcore_barrier()
    cps = [pltpu.async_copy(onesv, shared.at[didx.at[j]], ssem, add=True)
           for j in range(_NCH)]
    for cp in cps:
        cp.wait()
    plsc.subcore_barrier()
    pltpu.sync_copy(shared.at[pl.ds(s * _RPT, _RPT)], stage)
    pltpu.sync_copy(stage, out_hbm.at[c, pl.ds(s * _RPT, _RPT)])


_NBUF = 4


def _sc_y_body(src3, dst3, xws_hbm, zeros_hbm, out_hbm,
               sidx, didx, gbuf, stage, shared, gsem, ssem):
    c = lax.axis_index("c")
    s = lax.axis_index("s")
    gw = c * _NSUB + s
    for r in range(_RPT // _CHUNK):
        base = s * _RPT + r * _CHUNK
        pltpu.sync_copy(zeros_hbm.at[pl.ds(base, _CHUNK)], stage)
        pltpu.sync_copy(stage, shared.at[pl.ds(base, _CHUNK)])
    pltpu.sync_copy(src3.at[gw], sidx)
    pltpu.sync_copy(dst3.at[gw], didx)
    plsc.subcore_barrier()
    gcps = [None] * _NCH
    scps = [None] * _NCH
    for j in range(min(_NBUF, _NCH)):
        gcps[j] = pltpu.async_copy(xws_hbm.at[sidx.at[j]], gbuf.at[j], gsem)
    for j in range(_NCH):
        if j >= _NBUF:
            scps[j - _NBUF].wait()
            gcps[j] = pltpu.async_copy(xws_hbm.at[sidx.at[j]],
                                       gbuf.at[j % _NBUF], gsem)
        gcps[j].wait()
        scps[j] = pltpu.async_copy(gbuf.at[j % _NBUF], shared.at[didx.at[j]],
                                   ssem, add=True)
    for j in range(max(_NCH - _NBUF, 0), _NCH):
        scps[j].wait()
    plsc.subcore_barrier()
    for r in range(_RPT // _CHUNK):
        base = s * _RPT + r * _CHUNK
        pltpu.sync_copy(shared.at[pl.ds(base, _CHUNK)], stage)
        pltpu.sync_copy(stage, out_hbm.at[c, pl.ds(base, _CHUNK)])


def _softmax(x):
    m = jnp.max(x, axis=-1, keepdims=True)
    e = jnp.exp(x - m)
    return e / jnp.sum(e, axis=-1, keepdims=True)


def _sigmoid(x):
    return 1.0 / (1.0 + jnp.exp(-x))


def _silu(x):
    return x * _sigmoid(x)


def _gate(score, g):
    return _sigmoid(score + _softmax((score + g[0]) / _TAU)
                    - _softmax((score + g[1]) / _TAU))


def _top2_weights(logits):
    cols = jax.lax.broadcasted_iota(jnp.int32, logits.shape, 1)
    v1 = jnp.max(logits, axis=1, keepdims=True)
    i1 = jnp.min(jnp.where(logits == v1, cols, _EXP), axis=1, keepdims=True)
    m1 = cols == i1
    l2 = jnp.where(m1, -jnp.inf, logits)
    v2 = jnp.max(l2, axis=1, keepdims=True)
    i2 = jnp.min(jnp.where(l2 == v2, cols, _EXP), axis=1, keepdims=True)
    m2 = cols == i2
    w1 = _sigmoid(v1 - v2)
    return jnp.where(m1, w1, 0.0) + jnp.where(m2, 1.0 - w1, 0.0)


def _scale_body(emb_x_ref, gcn_w_ref, deg_ref, xws_ref):
    xw = jnp.dot(emb_x_ref[...].astype(jnp.bfloat16),
                 gcn_w_ref[...].astype(jnp.bfloat16),
                 preferred_element_type=jnp.float32)
    deg = jnp.sum(deg_ref[...], axis=0)[:, 0:1] + 1.0
    xws_ref[...] = xw * jax.lax.rsqrt(deg)


def _main_body(emb_x_ref, emb_v_ref, y_ref, deg_ref, xws_ref, gcn_b_ref,
               gn_ref, vs_ref, lw_ref, lb_ref, w1cat_ref, b1cat_ref,
               exw2_ref, exb2_ref, exg_ref, exbeta_ref,
               evxw2_ref, evxb2_ref, evw_ref, evg_ref,
               ln2g_ref, ln2b_ref, lnv2g_ref,
               lx_ref, lv_ref, score_ref):
    x = emb_x_ref[...]
    v = emb_v_ref[...]
    deg = jnp.sum(deg_ref[...], axis=0)[:, 0:1] + 1.0
    dinv = jax.lax.rsqrt(deg)
    ysum = jnp.sum(y_ref[...], axis=0) + xws_ref[...]
    gcn = dinv * ysum[:, :_EXP] + gcn_b_ref[...][0:1, :_EXP]
    score_x = _softmax(gcn)
    gn = gn_ref[...]
    logits_x = _gate(score_x, (gn[:, 0:8], gn[:, 8:16]))

    sv = jax.lax.dot_general(vs_ref[...], lw_ref[...],
                             (((1,), (0,)), ((), ())),
                             preferred_element_type=jnp.float32) \
        + lb_ref[...][0:1, :]
    score_v = _softmax(sv[:, :_EXP])
    logits_v = _gate(score_v, (gn[:, 16:24], gn[:, 24:32]))

    wx = _top2_weights(logits_x)
    wv = _top2_weights(logits_v)

    h0 = _silu(jnp.dot(x, w1cat_ref[...],
                       preferred_element_type=jnp.float32)
               + b1cat_ref[...][0:1, :])

    lx_acc = jnp.zeros((_BN, _H), jnp.float32)
    lv_acc = [jnp.zeros((_BN, _H), jnp.float32) for _ in range(3)]
    for e in range(_EXP):
        h1x = h0[:, e * _H:(e + 1) * _H]
        h2 = _silu(jnp.dot(h1x, exw2_ref[e],
                           preferred_element_type=jnp.float32)
                   + exb2_ref[e][None, :])
        mu = jnp.mean(h2, axis=1, keepdims=True)
        var = jnp.mean((h2 - mu) ** 2, axis=1, keepdims=True)
        aex = (h2 - mu) * jax.lax.rsqrt(var + _EPS_LN) \
            * exg_ref[e][None, :] + exbeta_ref[e][None, :]
        lx_acc = lx_acc + wx[:, e:e + 1] * aex

        h1v = h0[:, (_EXP + e) * _H:(_EXP + e + 1) * _H]
        hv2 = _silu(jnp.dot(h1v, evxw2_ref[e],
                            preferred_element_type=jnp.float32)
                    + evxb2_ref[e][None, :])
        ev = jnp.dot(hv2, evw_ref[e], preferred_element_type=jnp.float32)
        aev = ev[:, None, :] * v
        rms = jax.lax.rsqrt(
            jnp.mean(aev * aev, axis=(1, 2), keepdims=True) + _EPS_LN)
        aev = aev * rms * evg_ref[e][None, None, :]
        for k in range(3):
            c = e + _EXP * k
            lv_acc[c % 3] = lv_acc[c % 3] \
                + wv[:, c // 3:c // 3 + 1] * aev[:, k, :]

    lxs = lx_acc + x
    mu = jnp.mean(lxs, axis=1, keepdims=True)
    var = jnp.mean((lxs - mu) ** 2, axis=1, keepdims=True)
    lx_ref[...] = (lxs - mu) * jax.lax.rsqrt(var + _EPS_LN) \
        * ln2g_ref[...][0:1, :] + ln2b_ref[...][0:1, :]

    lvs = jnp.stack(lv_acc, axis=1) + v
    rms2 = jax.lax.rsqrt(
        jnp.mean(lvs * lvs, axis=(1, 2), keepdims=True) + _EPS_LN)
    lv_ref[...] = lvs * rms2 * lnv2g_ref[...][0:1, None, :]

    score_ref[...] = _softmax(
        jnp.concatenate([logits_x, logits_v], axis=1))


def _gumbel_noise():
    base = jax.random.key(42)
    gs = []
    for i in range(4):
        u = jax.random.uniform(jax.random.fold_in(base, i), (_N, _EXP),
                               minval=1e-6, maxval=1.0 - 1e-6)
        gs.append(-jnp.log(-jnp.log(u)))
    return jnp.concatenate(gs, axis=1)


def kernel(emb_x, emb_v, edge_index, gcn_W, gcn_b, lin_W, lin_b,
           exW1, exb1, exW2, exb2, exg, exbeta,
           evxW1, evxb1, evxW2, evxb2, evW, evg, ln2g, ln2b, lnv2g):
    src3 = edge_index[0].reshape(_NTILE, _NCH, _CHUNK)
    dst3 = edge_index[1].reshape(_NTILE, _NCH, _CHUNK)
    zeros_n = jnp.zeros((_N, _W), jnp.float32)
    ones_c = jnp.ones((_CHUNK, _W), jnp.float32)

    mesh = plsc.VectorSubcoreMesh(core_axis_name="c", subcore_axis_name="s")
    deg_p = pl.kernel(
        _sc_deg_body,
        out_type=jax.ShapeDtypeStruct((_NSC, _N, _W), jnp.float32),
        mesh=mesh,
        scratch_types=[
            pltpu.VMEM((_NCH, _CHUNK), jnp.int32),
            pltpu.VMEM((_CHUNK, _W), jnp.float32),
            pltpu.VMEM((_RPT, _W), jnp.float32),
            pltpu.VMEM_SHARED((_N, _W), jnp.float32),
            pltpu.SemaphoreType.DMA,
        ],
    )(dst3, zeros_n, ones_c)

    gcn_wp = jnp.pad(gcn_W, ((0, 0), (0, _W - _EXP)))
    xws = pl.pallas_call(
        _scale_body,
        grid=(1,),
        in_specs=[
            pl.BlockSpec((_N, _H), lambda i: (0, 0)),
            pl.BlockSpec((_H, _W), lambda i: (0, 0)),
            pl.BlockSpec((_NSC, _N, _W), lambda i: (0, 0, 0)),
        ],
        out_specs=pl.BlockSpec((_N, _W), lambda i: (0, 0)),
        out_shape=jax.ShapeDtypeStruct((_N, _W), jnp.float32),
        interpret=_INTERPRET,
    )(emb_x, gcn_wp, deg_p)

    y_p = pl.kernel(
        _sc_y_body,
        out_type=jax.ShapeDtypeStruct((_NSC, _N, _W), jnp.float32),
        mesh=mesh,
        scratch_types=[
            pltpu.VMEM((_NCH, _CHUNK), jnp.int32),
            pltpu.VMEM((_NCH, _CHUNK), jnp.int32),
            pltpu.VMEM((_NBUF, _CHUNK, _W), jnp.float32),
            pltpu.VMEM((_CHUNK, _W), jnp.float32),
            pltpu.VMEM_SHARED((_N, _W), jnp.float32),
            pltpu.SemaphoreType.DMA,
            pltpu.SemaphoreType.DMA,
        ],
    )(src3, dst3, xws, zeros_n)

    v_sum = jnp.einsum('nai,naj->nij', emb_v, emb_v,
                       preferred_element_type=jnp.float32,
                       precision=jax.lax.Precision.DEFAULT)
    vs16 = v_sum.astype(jnp.bfloat16).reshape(_N, _H * _H)

    lin_wp = jnp.pad(lin_W, ((0, 0), (0, 16 - _EXP)))
    lin_bp = jnp.pad(lin_b, (0, 16 - _EXP))[None, :]
    gnoise = _gumbel_noise()
    w1cat = jnp.concatenate(
        [jnp.transpose(exW1, (1, 0, 2)).reshape(_H, _EXP * _H),
         jnp.transpose(evxW1, (1, 0, 2)).reshape(_H, _EXP * _H)], axis=1)
    b1cat = jnp.concatenate([exb1.reshape(-1), evxb1.reshape(-1)])[None, :]
    gcn_bp = jnp.pad(gcn_b, (0, 16 - _EXP))[None, :]

    grid = (_N // _BN,)
    full = lambda *s: pl.BlockSpec(s, lambda i: (0,) * len(s))
    lx, lv, score = pl.pallas_call(
        _main_body,
        grid=grid,
        in_specs=[
            pl.BlockSpec((_BN, _H), lambda i: (i, 0)),
            pl.BlockSpec((_BN, 3, _H), lambda i: (i, 0, 0)),
            pl.BlockSpec((_NSC, _BN, _W), lambda i: (0, i, 0)),
            pl.BlockSpec((_NSC, _BN, _W), lambda i: (0, i, 0)),
            pl.BlockSpec((_BN, _W), lambda i: (i, 0)),
            full(1, 16),
            pl.BlockSpec((_BN, 32), lambda i: (i, 0)),
            pl.BlockSpec((_BN, _H * _H), lambda i: (i, 0)),
            full(_H * _H, 16),
            full(1, 16),
            full(_H, 2 * _EXP * _H),
            full(1, 2 * _EXP * _H),
            full(_EXP, _H, _H),
            full(_EXP, _H),
            full(_EXP, _H),
            full(_EXP, _H),
            full(_EXP, _H, _H),
            full(_EXP, _H),
            full(_EXP, _H, _H),
            full(_EXP, _H),
            full(1, _H),
            full(1, _H),
            full(1, _H),
        ],
        out_specs=[
            pl.BlockSpec((_BN, _H), lambda i: (i, 0)),
            pl.BlockSpec((_BN, 3, _H), lambda i: (i, 0, 0)),
            pl.BlockSpec((_BN, 16), lambda i: (i, 0)),
        ],
        out_shape=[
            jax.ShapeDtypeStruct((_N, _H), jnp.float32),
            jax.ShapeDtypeStruct((_N, 3, _H), jnp.float32),
            jax.ShapeDtypeStruct((_N, 16), jnp.float32),
        ],
        interpret=_INTERPRET,
    )(emb_x, emb_v, y_p, deg_p, xws, gcn_bp, gnoise, vs16, lin_wp, lin_bp,
      w1cat, b1cat, exW2, exb2, exg, exbeta, evxW2, evxb2, evW, evg,
      ln2g[None, :], ln2b[None, :], lnv2g[None, :])

    return lx, lv, score[:, :2 * _EXP]


# speedup vs baseline: 1.7762x; 1.2619x over previous
import functools

import jax
import jax.numpy as jnp
from jax import lax
from jax.experimental import pallas as pl
from jax.experimental.pallas import tpu as pltpu
from jax.experimental.pallas import tpu_sc as plsc

_N = 4096
_H = 128
_EXP = 8
_K = 2
_NE = 65536
_TAU = 1.0
_BN = 256
_SUB = 64
_EPS_LN = 1e-5

_NSC = 2
_NSUB = 16
_NTILE = _NSC * _NSUB
_EPT = _NE // _NTILE
_CHUNK = 128
_NCH = _EPT // _CHUNK
_RPT = _N // _NSUB
_W = 128

_INTERPRET = False


def _sc_deg_body(dst3, zeros_hbm, ones_hbm, out_hbm,
                 didx, onesv, stage, shared, ssem):
    c = lax.axis_index("c")
    s = lax.axis_index("s")
    gw = c * _NSUB + s
    pltpu.sync_copy(zeros_hbm.at[pl.ds(s * _RPT, _RPT)], stage)
    pltpu.sync_copy(stage, shared.at[pl.ds(s * _RPT, _RPT)])
    pltpu.sync_copy(ones_hbm, onesv)
    pltpu.sync_copy(dst3.at[gw], didx)
    plsc.subcore_barrier()
    cps = [pltpu.async_copy(onesv, shared.at[didx.at[j]], ssem, add=True)
           for j in range(_NCH)]
    for cp in cps:
        cp.wait()
    plsc.subcore_barrier()
    pltpu.sync_copy(shared.at[pl.ds(s * _RPT, _RPT)], stage)
    pltpu.sync_copy(stage, out_hbm.at[c, pl.ds(s * _RPT, _RPT)])


_NBUF = 4


def _sc_y_body(src3, dst3, xws_hbm, zeros_hbm, out_hbm,
               sidx, didx, gbuf, stage, shared, gsem, ssem):
    c = lax.axis_index("c")
    s = lax.axis_index("s")
    gw = c * _NSUB + s
    for r in range(_RPT // _CHUNK):
        base = s * _RPT + r * _CHUNK
        pltpu.sync_copy(zeros_hbm.at[pl.ds(base, _CHUNK)], stage)
        pltpu.sync_copy(stage, shared.at[pl.ds(base, _CHUNK)])
    pltpu.sync_copy(src3.at[gw], sidx)
    pltpu.sync_copy(dst3.at[gw], didx)
    plsc.subcore_barrier()
    gcps = [None] * _NCH
    scps = [None] * _NCH
    for j in range(min(_NBUF, _NCH)):
        gcps[j] = pltpu.async_copy(xws_hbm.at[sidx.at[j]], gbuf.at[j], gsem)
    for j in range(_NCH):
        if j >= _NBUF:
            scps[j - _NBUF].wait()
            gcps[j] = pltpu.async_copy(xws_hbm.at[sidx.at[j]],
                                       gbuf.at[j % _NBUF], gsem)
        gcps[j].wait()
        scps[j] = pltpu.async_copy(gbuf.at[j % _NBUF], shared.at[didx.at[j]],
                                   ssem, add=True)
    for j in range(max(_NCH - _NBUF, 0), _NCH):
        scps[j].wait()
    plsc.subcore_barrier()
    for r in range(_RPT // _CHUNK):
        base = s * _RPT + r * _CHUNK
        pltpu.sync_copy(shared.at[pl.ds(base, _CHUNK)], stage)
        pltpu.sync_copy(stage, out_hbm.at[c, pl.ds(base, _CHUNK)])


def _softmax(x):
    m = jnp.max(x, axis=-1, keepdims=True)
    e = jnp.exp(x - m)
    return e / jnp.sum(e, axis=-1, keepdims=True)


def _sigmoid(x):
    return 1.0 / (1.0 + jnp.exp(-x))


def _silu(x):
    return x * _sigmoid(x)


def _gate(score, g):
    return _sigmoid(score + _softmax((score + g[0]) / _TAU)
                    - _softmax((score + g[1]) / _TAU))


def _top2_weights(logits):
    cols = jax.lax.broadcasted_iota(jnp.int32, logits.shape, 1)
    v1 = jnp.max(logits, axis=1, keepdims=True)
    i1 = jnp.min(jnp.where(logits == v1, cols, _EXP), axis=1, keepdims=True)
    m1 = cols == i1
    l2 = jnp.where(m1, -jnp.inf, logits)
    v2 = jnp.max(l2, axis=1, keepdims=True)
    i2 = jnp.min(jnp.where(l2 == v2, cols, _EXP), axis=1, keepdims=True)
    m2 = cols == i2
    w1 = _sigmoid(v1 - v2)
    return jnp.where(m1, w1, 0.0) + jnp.where(m2, 1.0 - w1, 0.0)


def _scale_body(emb_x_ref, gcn_w_ref, deg_ref, xws_ref):
    xw = jnp.dot(emb_x_ref[...].astype(jnp.bfloat16),
                 gcn_w_ref[...].astype(jnp.bfloat16),
                 preferred_element_type=jnp.float32)
    deg = jnp.sum(deg_ref[...], axis=0)[:, 0:1] + 1.0
    xws_ref[...] = xw * jax.lax.rsqrt(deg)


def _main_body(emb_x_ref, emb_v_ref, y_ref, deg_ref, xws_ref, gcn_b_ref,
               gn_ref, vs_ref, lw_ref, lb_ref, w1cat_ref, b1cat_ref,
               exw2_ref, exb2_ref, exg_ref, exbeta_ref,
               evxw2_ref, evxb2_ref, evw_ref, evg_ref,
               ln2g_ref, ln2b_ref, lnv2g_ref,
               lx_ref, lv_ref, score_ref):
    x = emb_x_ref[...]
    vp = [emb_v_ref[k] for k in range(3)]
    deg = jnp.sum(deg_ref[...], axis=0)[:, 0:1] + 1.0
    dinv = jax.lax.rsqrt(deg)
    ysum = jnp.sum(y_ref[...], axis=0) + xws_ref[...]
    gcn = dinv * ysum[:, :_EXP] + gcn_b_ref[...][0:1, :_EXP]
    score_x = _softmax(gcn)
    gn = gn_ref[...]
    logits_x = _gate(score_x, (gn[:, 0:8], gn[:, 8:16]))

    sv = jax.lax.dot_general(vs_ref[...], lw_ref[...],
                             (((1,), (0,)), ((), ())),
                             preferred_element_type=jnp.float32) \
        + lb_ref[...][0:1, :]
    score_v = _softmax(sv[:, :_EXP])
    logits_v = _gate(score_v, (gn[:, 16:24], gn[:, 24:32]))

    wx = _top2_weights(logits_x)
    wv = _top2_weights(logits_v)

    h0 = _silu(jnp.dot(x, w1cat_ref[...],
                       preferred_element_type=jnp.float32)
               + b1cat_ref[...][0:1, :])

    lx_acc = jnp.zeros((_BN, _H), jnp.float32)
    lv_acc = [jnp.zeros((_BN, _H), jnp.float32) for _ in range(3)]
    for e in range(_EXP):
        h1x = h0[:, e * _H:(e + 1) * _H]
        h2 = _silu(jnp.dot(h1x, exw2_ref[e],
                           preferred_element_type=jnp.float32)
                   + exb2_ref[e][None, :])
        mu = jnp.mean(h2, axis=1, keepdims=True)
        var = jnp.mean((h2 - mu) ** 2, axis=1, keepdims=True)
        aex = (h2 - mu) * jax.lax.rsqrt(var + _EPS_LN) \
            * exg_ref[e][None, :] + exbeta_ref[e][None, :]
        lx_acc = lx_acc + wx[:, e:e + 1] * aex

        h1v = h0[:, (_EXP + e) * _H:(_EXP + e + 1) * _H]
        hv2 = _silu(jnp.dot(h1v, evxw2_ref[e],
                            preferred_element_type=jnp.float32)
                    + evxb2_ref[e][None, :])
        ev = jnp.dot(hv2, evw_ref[e], preferred_element_type=jnp.float32)
        aev = [ev * vp[k] for k in range(3)]
        ss = sum(jnp.sum(a * a, axis=1, keepdims=True) for a in aev)
        rms = jax.lax.rsqrt(ss / (3.0 * _H) + _EPS_LN)
        for k in range(3):
            c = e + _EXP * k
            lv_acc[c % 3] = lv_acc[c % 3] \
                + wv[:, c // 3:c // 3 + 1] * (aev[k] * rms
                                              * evg_ref[e][None, :])

    lxs = lx_acc + x
    mu = jnp.mean(lxs, axis=1, keepdims=True)
    var = jnp.mean((lxs - mu) ** 2, axis=1, keepdims=True)
    lx_ref[...] = (lxs - mu) * jax.lax.rsqrt(var + _EPS_LN) \
        * ln2g_ref[...][0:1, :] + ln2b_ref[...][0:1, :]

    lvs = [lv_acc[k] + vp[k] for k in range(3)]
    ss2 = sum(jnp.sum(a * a, axis=1, keepdims=True) for a in lvs)
    rms2 = jax.lax.rsqrt(ss2 / (3.0 * _H) + _EPS_LN)
    for k in range(3):
        lv_ref[k] = lvs[k] * rms2 * lnv2g_ref[...][0:1, :]

    score_ref[...] = _softmax(
        jnp.concatenate([logits_x, logits_v], axis=1))


def _gumbel_noise():
    base = jax.random.key(42)
    gs = []
    for i in range(4):
        u = jax.random.uniform(jax.random.fold_in(base, i), (_N, _EXP),
                               minval=1e-6, maxval=1.0 - 1e-6)
        gs.append(-jnp.log(-jnp.log(u)))
    return jnp.concatenate(gs, axis=1)


def kernel(emb_x, emb_v, edge_index, gcn_W, gcn_b, lin_W, lin_b,
           exW1, exb1, exW2, exb2, exg, exbeta,
           evxW1, evxb1, evxW2, evxb2, evW, evg, ln2g, ln2b, lnv2g):
    src3 = edge_index[0].reshape(_NTILE, _NCH, _CHUNK)
    dst3 = edge_index[1].reshape(_NTILE, _NCH, _CHUNK)
    zeros_n = jnp.zeros((_N, _W), jnp.float32)
    ones_c = jnp.ones((_CHUNK, _W), jnp.float32)

    mesh = plsc.VectorSubcoreMesh(core_axis_name="c", subcore_axis_name="s")
    deg_p = pl.kernel(
        _sc_deg_body,
        out_type=jax.ShapeDtypeStruct((_NSC, _N, _W), jnp.float32),
        mesh=mesh,
        scratch_types=[
            pltpu.VMEM((_NCH, _CHUNK), jnp.int32),
            pltpu.VMEM((_CHUNK, _W), jnp.float32),
            pltpu.VMEM((_RPT, _W), jnp.float32),
            pltpu.VMEM_SHARED((_N, _W), jnp.float32),
            pltpu.SemaphoreType.DMA,
        ],
    )(dst3, zeros_n, ones_c)

    gcn_wp = jnp.pad(gcn_W, ((0, 0), (0, _W - _EXP)))
    xws = pl.pallas_call(
        _scale_body,
        grid=(1,),
        in_specs=[
            pl.BlockSpec((_N, _H), lambda i: (0, 0)),
            pl.BlockSpec((_H, _W), lambda i: (0, 0)),
            pl.BlockSpec((_NSC, _N, _W), lambda i: (0, 0, 0)),
        ],
        out_specs=pl.BlockSpec((_N, _W), lambda i: (0, 0)),
        out_shape=jax.ShapeDtypeStruct((_N, _W), jnp.float32),
        interpret=_INTERPRET,
    )(emb_x, gcn_wp, deg_p)

    y_p = pl.kernel(
        _sc_y_body,
        out_type=jax.ShapeDtypeStruct((_NSC, _N, _W), jnp.float32),
        mesh=mesh,
        scratch_types=[
            pltpu.VMEM((_NCH, _CHUNK), jnp.int32),
            pltpu.VMEM((_NCH, _CHUNK), jnp.int32),
            pltpu.VMEM((_NBUF, _CHUNK, _W), jnp.float32),
            pltpu.VMEM((_CHUNK, _W), jnp.float32),
            pltpu.VMEM_SHARED((_N, _W), jnp.float32),
            pltpu.SemaphoreType.DMA,
            pltpu.SemaphoreType.DMA,
        ],
    )(src3, dst3, xws, zeros_n)

    v_sum = jnp.einsum('nai,naj->nij', emb_v, emb_v,
                       preferred_element_type=jnp.float32,
                       precision=jax.lax.Precision.DEFAULT)
    vs16 = v_sum.astype(jnp.bfloat16).reshape(_N, _H * _H)

    lin_wp = jnp.pad(lin_W, ((0, 0), (0, 16 - _EXP)))
    lin_bp = jnp.pad(lin_b, (0, 16 - _EXP))[None, :]
    gnoise = _gumbel_noise()
    w1cat = jnp.concatenate(
        [jnp.transpose(exW1, (1, 0, 2)).reshape(_H, _EXP * _H),
         jnp.transpose(evxW1, (1, 0, 2)).reshape(_H, _EXP * _H)], axis=1)
    b1cat = jnp.concatenate([exb1.reshape(-1), evxb1.reshape(-1)])[None, :]
    gcn_bp = jnp.pad(gcn_b, (0, 16 - _EXP))[None, :]

    grid = (_N // _BN,)
    full = lambda *s: pl.BlockSpec(s, lambda i: (0,) * len(s))
    lx, lv, score = pl.pallas_call(
        _main_body,
        grid=grid,
        in_specs=[
            pl.BlockSpec((_BN, _H), lambda i: (i, 0)),
            pl.BlockSpec((3, _BN, _H), lambda i: (0, i, 0)),
            pl.BlockSpec((_NSC, _BN, _W), lambda i: (0, i, 0)),
            pl.BlockSpec((_NSC, _BN, _W), lambda i: (0, i, 0)),
            pl.BlockSpec((_BN, _W), lambda i: (i, 0)),
            full(1, 16),
            pl.BlockSpec((_BN, 32), lambda i: (i, 0)),
            pl.BlockSpec((_BN, _H * _H), lambda i: (i, 0)),
            full(_H * _H, 16),
            full(1, 16),
            full(_H, 2 * _EXP * _H),
            full(1, 2 * _EXP * _H),
            full(_EXP, _H, _H),
            full(_EXP, _H),
            full(_EXP, _H),
            full(_EXP, _H),
            full(_EXP, _H, _H),
            full(_EXP, _H),
            full(_EXP, _H, _H),
            full(_EXP, _H),
            full(1, _H),
            full(1, _H),
            full(1, _H),
        ],
        out_specs=[
            pl.BlockSpec((_BN, _H), lambda i: (i, 0)),
            pl.BlockSpec((3, _BN, _H), lambda i: (0, i, 0)),
            pl.BlockSpec((_BN, 16), lambda i: (i, 0)),
        ],
        out_shape=[
            jax.ShapeDtypeStruct((_N, _H), jnp.float32),
            jax.ShapeDtypeStruct((3, _N, _H), jnp.float32),
            jax.ShapeDtypeStruct((_N, 16), jnp.float32),
        ],
        interpret=_INTERPRET,
    )(emb_x, jnp.transpose(emb_v, (1, 0, 2)), y_p, deg_p, xws, gcn_bp,
      gnoise, vs16, lin_wp, lin_bp,
      w1cat, b1cat, exW2, exb2, exg, exbeta, evxW2, evxb2, evW, evg,
      ln2g[None, :], ln2b[None, :], lnv2g[None, :])

    return lx, jnp.transpose(lv, (1, 0, 2)), score[:, :2 * _EXP]
